# Initial kernel scaffold; baseline (speedup 1.0000x reference)
import jax
import jax.numpy as jnp
from jax.experimental import pallas as pl


def kernel(x, Wr, br, W1, b1, W2, b2):
    raise NotImplementedError("write your pallas kernel here")


# speedup vs baseline: 2.9022x; 2.9022x over previous
import functools

import jax
import jax.numpy as jnp
from jax.experimental import pallas as pl
from jax.experimental.pallas import tpu as pltpu

D_MODEL_ = 768
D_FF_ = 3072
N_EXP_ = 8
N_TOK_ = 2048
LANES_ = 128


def _gelu_exact(h):
    return h * 0.5 * (1.0 + jax.lax.erf(h * jnp.float32(0.7071067811865476)))


def _router_combine(xt, wr_pad, br_pad):
    lane = jax.lax.broadcasted_iota(jnp.int32, (TOK_BLK_, LANES_), 1)
    valid = lane < N_EXP_
    logits = jax.lax.dot_general(
        xt, wr_pad, (((1,), (0,)), ((), ())),
        preferred_element_type=jnp.float32) + br_pad
    neg = jnp.float32(-1e30)
    logits = jnp.where(valid, logits, neg)
    m = jnp.max(logits, axis=1, keepdims=True)
    e = jnp.where(valid, jnp.exp(logits - m), 0.0)
    s = jnp.sum(e, axis=1, keepdims=True)
    probs = e / s
    m1 = jnp.max(probs, axis=1, keepdims=True)
    i1 = jnp.min(jnp.where(probs == m1, lane, N_EXP_), axis=1, keepdims=True)
    sel1 = lane == i1
    probs2 = jnp.where(sel1, -1.0, probs)
    m2 = jnp.max(probs2, axis=1, keepdims=True)
    i2 = jnp.min(jnp.where(probs2 == m2, lane, N_EXP_), axis=1, keepdims=True)
    sel2 = lane == i2
    tot = jnp.clip(m1 + m2, 1e-9, None)
    return (jnp.where(sel1, m1, 0.0) + jnp.where(sel2, m2, 0.0)) / tot

TOK_BLK_ = 512


def _moe_dense_kernel(xt_ref, wr_ref, br_ref, w1_ref, b1_ref, w2_ref, b2_ref,
                      out_ref, comb_ref):
    e = pl.program_id(1)

    @pl.when(e == 0)
    def _():
        comb_ref[...] = _router_combine(xt_ref[...], wr_ref[...], br_ref[...])
        out_ref[...] = jnp.zeros_like(out_ref)

    lane = jax.lax.broadcasted_iota(jnp.int32, (TOK_BLK_, LANES_), 1)
    w_col = jnp.sum(jnp.where(lane == e, comb_ref[...], 0.0), axis=1,
                    keepdims=True)
    h = jax.lax.dot_general(
        xt_ref[...], w1_ref[0], (((1,), (0,)), ((), ())),
        preferred_element_type=jnp.float32) + b1_ref[0]
    h = _gelu_exact(h)
    y = jax.lax.dot_general(
        h, w2_ref[0], (((1,), (0,)), ((), ())),
        preferred_element_type=jnp.float32) + b2_ref[0]
    out_ref[...] += w_col * y


@jax.jit
def kernel(x, Wr, br, W1, b1, W2, b2):
    orig_shape = x.shape
    xt = x.reshape(-1, D_MODEL_)
    wr_pad = jnp.zeros((D_MODEL_, LANES_), jnp.float32).at[:, :N_EXP_].set(Wr)
    br_pad = jnp.zeros((LANES_,), jnp.float32).at[:N_EXP_].set(br)

    out = pl.pallas_call(
        _moe_dense_kernel,
        grid=(N_TOK_ // TOK_BLK_, N_EXP_),
        in_specs=[
            pl.BlockSpec((TOK_BLK_, D_MODEL_), lambda t, e: (t, 0)),
            pl.BlockSpec((D_MODEL_, LANES_), lambda t, e: (0, 0)),
            pl.BlockSpec((LANES_,), lambda t, e: (0,)),
            pl.BlockSpec((1, D_MODEL_, D_FF_), lambda t, e: (e, 0, 0)),
            pl.BlockSpec((1, 1, D_FF_), lambda t, e: (e, 0, 0)),
            pl.BlockSpec((1, D_FF_, D_MODEL_), lambda t, e: (e, 0, 0)),
            pl.BlockSpec((1, 1, D_MODEL_), lambda t, e: (e, 0, 0)),
        ],
        out_specs=pl.BlockSpec((TOK_BLK_, D_MODEL_), lambda t, e: (t, 0)),
        out_shape=jax.ShapeDtypeStruct((N_TOK_, D_MODEL_), jnp.float32),
        scratch_shapes=[pltpu.VMEM((TOK_BLK_, LANES_), jnp.float32)],
    )(xt, wr_pad, br_pad, W1, b1.reshape(N_EXP_, 1, D_FF_), W2,
      b2.reshape(N_EXP_, 1, D_MODEL_))
    return out.reshape(orig_shape)
